# baseline (speedup 1.0000x reference)
import functools

import jax
import jax.numpy as jnp
from jax import lax
from jax.experimental import pallas as pl
from jax.experimental.pallas import tpu as pltpu
from jax.experimental.pallas import tpu_sc as plsc

N_TOTAL = 50000
N_PAD = 50176
HALF = 32
E_TOTAL = 800000
E_PAD = 819200
GROUP = 128
GROUPS_PER_SUPER = 4
SUPER = GROUP * GROUPS_PER_SUPER
N_SUPER = E_PAD // SUPER
N_GROUPS = E_PAD // GROUP
N_SUB = 16
SUPERS_PER_TILE = N_SUPER // N_SUB
RPT = N_PAD // N_SUB

_MESH = plsc.VectorSubcoreMesh(core_axis_name="c", subcore_axis_name="s")


@functools.partial(
    pl.kernel,
    out_type=jax.ShapeDtypeStruct((2 * N_PAD, HALF), jnp.float32),
    mesh=_MESH,
    scratch_types=[
        pltpu.VMEM((GROUPS_PER_SUPER, GROUP), jnp.int32),
        pltpu.VMEM((GROUPS_PER_SUPER, GROUP), jnp.int32),
        pltpu.VMEM((GROUPS_PER_SUPER, GROUP), jnp.int32),
        pltpu.VMEM((GROUPS_PER_SUPER, GROUP), jnp.int32),
        pltpu.VMEM((SUPER, 16), jnp.float32),
        pltpu.VMEM((SUPER, 16), jnp.float32),
        pltpu.VMEM((GROUP, HALF), jnp.float32),
        pltpu.VMEM((GROUP, HALF), jnp.float32),
        pltpu.VMEM_SHARED((N_PAD, HALF), jnp.float32),
        pltpu.SemaphoreType.DMA,
        pltpu.SemaphoreType.DMA,
        pltpu.SemaphoreType.DMA,
        pltpu.SemaphoreType.DMA,
    ],
    compiler_params=pltpu.CompilerParams(use_tc_tiling_on_sc=False),
)
def _spmm_layer(x_h, src_h, dst_h, w_h, zero_h, y_h,
                idx_s0, idx_s1, idx_d0, idx_d1, w16_0, w16_1,
                rows0, rows1, acc, gsem0, gsem1, ssem0, ssem1):
    c = lax.axis_index("c")
    s = lax.axis_index("s")

    idx_s = (idx_s0, idx_s1)
    idx_d = (idx_d0, idx_d1)
    w16 = (w16_0, w16_1)
    rows = (rows0, rows1)
    gsem = (gsem0, gsem1)
    ssem = (ssem0, ssem1)

    pltpu.sync_copy(zero_h, acc.at[pl.ds(s * RPT, RPT)])
    plsc.subcore_barrier()

    def stage(t, p, issue):
        j = jnp.minimum(s + t * N_SUB, N_SUPER - 1)
        cp1 = pltpu.make_async_copy(
            src_h.at[pl.ds(c * N_GROUPS + j * GROUPS_PER_SUPER,
                           GROUPS_PER_SUPER)], idx_s[p], ssem[p])
        cp2 = pltpu.make_async_copy(
            dst_h.at[pl.ds(j * GROUPS_PER_SUPER, GROUPS_PER_SUPER)],
            idx_d[p], ssem[p])
        cp3 = pltpu.make_async_copy(
            w_h.at[pl.ds(j * SUPER, SUPER)], w16[p], ssem[p])
        if issue:
            cp1.start()
            cp2.start()
            cp3.start()
        else:
            cp1.wait()
            cp2.wait()
            cp3.wait()

    def process_super(p):
        h = pltpu.make_async_copy(x_h.at[idx_s[p].at[0]], rows[0], gsem[0])
        h.start()
        for g in range(GROUPS_PER_SUPER):
            rb = rows[g % 2]
            if g + 1 < GROUPS_PER_SUPER:
                nh = pltpu.make_async_copy(
                    x_h.at[idx_s[p].at[g + 1]], rows[(g + 1) % 2],
                    gsem[(g + 1) % 2])
                nh.start()
            h.wait()
            h = nh if g + 1 < GROUPS_PER_SUPER else None

            base = g * GROUP

            def scale(k4, cc):
                for u in range(4):
                    k = k4 * 4 + u
                    wv = w16[p][base + k, pl.ds(0, 16)]
                    rb[k, pl.ds(0, 16)] = rb[k, pl.ds(0, 16)] * wv
                    rb[k, pl.ds(16, 16)] = rb[k, pl.ds(16, 16)] * wv
                return cc

            lax.fori_loop(0, GROUP // 4, scale, 0)

            pltpu.sync_copy(rb, acc.at[idx_d[p].at[g]], add=True)

    stage(0, 0, True)
    stage(1, 1, True)

    def outer(i, carry):
        t2 = i * 2
        for b in range(2):
            stage(t2 + b, b, False)
            process_super(b)
            stage(t2 + b + 2, b, True)
        return carry

    lax.fori_loop(0, SUPERS_PER_TILE // 2, outer, 0)
    stage(SUPERS_PER_TILE, 0, False)
    stage(SUPERS_PER_TILE + 1, 1, False)
    plsc.subcore_barrier()

    pltpu.sync_copy(acc.at[pl.ds(s * RPT, RPT)],
                    y_h.at[pl.ds(c * N_PAD + s * RPT, RPT)])


def _mean4(a, b, c, d):
    def body(ar, br, cr, dr, o):
        o[...] = (ar[...] + br[...] + cr[...] + dr[...]) * 0.25

    blk = 896
    nrow = a.shape[0]
    spec = pl.BlockSpec((blk, 128), lambda i: (i, 0))
    return pl.pallas_call(
        body,
        out_shape=jax.ShapeDtypeStruct(a.shape, jnp.float32),
        grid=(nrow // blk,),
        in_specs=[spec] * 4,
        out_specs=spec,
    )(a, b, c, d)


def kernel(edge_index, edge_weight, user_emb, item_emb):
    n_u = user_emb.shape[0]
    n = n_u + item_emb.shape[0]
    assert n == N_TOTAL and edge_weight.shape[0] == E_TOTAL

    all_emb = jnp.concatenate([user_emb, item_emb], axis=0)
    rpad = jnp.zeros((N_PAD - n, HALF), jnp.float32)
    x = jnp.concatenate(
        [all_emb[:, :HALF], rpad, all_emb[:, HALF:], rpad], axis=0)

    dst = edge_index[0].astype(jnp.int32)
    src = edge_index[1].astype(jnp.int32)
    n_pad = E_PAD - E_TOTAL
    pad_idx = (jnp.arange(n_pad, dtype=jnp.int32) * 64) % n
    src = jnp.concatenate([src, pad_idx])
    dst = jnp.concatenate([dst, pad_idx])
    w_pad = jnp.concatenate([edge_weight, jnp.zeros((n_pad,), jnp.float32)])
    src_big = jnp.concatenate([src, src + N_PAD]).reshape(2 * N_GROUPS, GROUP)
    dst2 = dst.reshape(N_GROUPS, GROUP)
    w16 = jnp.broadcast_to(w_pad[:, None], (E_PAD, 16))
    zeros = jnp.zeros((RPT, HALF), jnp.float32)

    xs = [x]
    for _ in range(3):
        x = _spmm_layer(x, src_big, dst2, w16, zeros)
        xs.append(x)

    xr = [v.reshape(2 * N_PAD * HALF // 128, 128) for v in xs]
    m = _mean4(*xr).reshape(2 * N_PAD, HALF)

    user_all = jnp.concatenate([m[:n_u], m[N_PAD:N_PAD + n_u]], axis=1)
    item_all = jnp.concatenate([m[n_u:n], m[N_PAD + n_u:N_PAD + n]], axis=1)
    return user_all, item_all
